# baseline (speedup 1.0000x reference)
import jax
import jax.numpy as jnp
from jax import lax
from jax.experimental import pallas as pl
from jax.experimental.pallas import tpu as pltpu
from jax.experimental.pallas import tpu_sc as plsc

VOCAB = 1000000
EMB = 64
OUT = 64
B = 4096
L = 200

NC = 2
NS = 16
NW = NC * NS
E_PER_W = B // NW
IDX_PER_W = E_PER_W * L

CHUNKS = ((0, 120), (120, 80))
UNROLL = 8


def _pool_body(x_hbm, table_hbm, out_hbm, idx_v, off_v, rows_v, pooled_v,
               sem0, sem1):
    c = lax.axis_index("c")
    s = lax.axis_index("s")
    wid = s * NC + c
    base_e = wid * E_PER_W
    sems = (sem0, sem1)
    last = jnp.int32(E_PER_W - 1)

    pltpu.sync_copy(x_hbm.at[pl.ds(base_e * L, IDX_PER_W)], idx_v)

    def prep(k, carry):
        for u in range(4):
            sl = pl.ds((k * 4 + u) * 16, 16)
            v = idx_v[sl]
            off_v[sl] = (v & 1) << 6
            idx_v[sl] = v >> 1
        return carry

    lax.fori_loop(0, IDX_PER_W // 64, prep, 0)

    def fire(e, buf):
        for off, n in CHUNKS:
            pltpu.async_copy(
                table_hbm.at[idx_v.at[pl.ds(e * L + off, n)]],
                rows_v.at[buf].at[pl.ds(off, n)],
                sems[buf],
            )

    def wait(buf):
        for off, n in CHUNKS:
            pltpu.make_async_copy(
                table_hbm.at[idx_v.at[pl.ds(off, n)]],
                rows_v.at[buf].at[pl.ds(off, n)],
                sems[buf],
            ).wait()

    zero = jnp.zeros((16,), jnp.float32)

    def reduce_into(e, buf):
        ebase = e * L

        def rows16(r, offv, nrows, accs):
            out = list(accs)
            for rr in range(nrows):
                half = offv[rr]
                for j in range(4):
                    out[j] = out[j] + rows_v[buf, r + rr,
                                             pl.ds(half + j * 16, 16)]
            return tuple(out)

        def red(i, accs):
            r = i * 16
            offv = off_v[pl.ds(ebase + r, 16)]
            return rows16(r, offv, 16, accs)

        acc = lax.fori_loop(0, (L // 16), red, (zero,) * 4, unroll=1)
        offv = off_v[pl.ds(ebase + (L // 16) * 16, 16)]
        acc = rows16((L // 16) * 16, offv, L - (L // 16) * 16, acc)
        scale = jnp.float32(1.0 / L)
        for j in range(4):
            pooled_v[e, pl.ds(j * 16, 16)] = acc[j] * scale
            pooled_v[e, pl.ds(64 + j * 16, 16)] = zero

    fire(jnp.int32(0), 0)
    fire(jnp.int32(1), 1)

    def pair(i, carry):
        e0 = 2 * i
        wait(0)
        reduce_into(e0, 0)
        fire(jnp.minimum(e0 + 2, last), 0)
        wait(1)
        reduce_into(e0 + 1, 1)
        fire(jnp.minimum(e0 + 3, last), 1)
        return carry

    lax.fori_loop(0, E_PER_W // 2, pair, 0)
    wait(0)
    wait(1)
    pltpu.sync_copy(pooled_v, out_hbm.at[pl.ds(base_e, E_PER_W)])


@jax.jit
def _pool(x_flat, table2):
    mesh = plsc.VectorSubcoreMesh(core_axis_name="c", subcore_axis_name="s")
    return pl.kernel(
        _pool_body,
        out_type=jax.ShapeDtypeStruct((B, 2 * EMB), jnp.float32),
        mesh=mesh,
        scratch_types=[
            pltpu.VMEM((IDX_PER_W,), jnp.int32),
            pltpu.VMEM((IDX_PER_W + 16,), jnp.int32),
            pltpu.VMEM((2, L, 2 * EMB), jnp.float32),
            pltpu.VMEM((E_PER_W, 2 * EMB), jnp.float32),
            pltpu.SemaphoreType.DMA,
            pltpu.SemaphoreType.DMA,
        ],
    )(x_flat, table2)


def _mm_body(p_ref, w_ref, b_ref, o_ref):
    o_ref[...] = (
        jnp.dot(p_ref[:, :EMB], w_ref[...], preferred_element_type=jnp.float32)
        + b_ref[...]
    )


def _matmul(pooled, W, b):
    return pl.pallas_call(
        _mm_body,
        out_shape=jax.ShapeDtypeStruct((B, OUT), jnp.float32),
    )(pooled, W, b.reshape(1, OUT))


def kernel(x, table, W, b):
    x_flat = x.reshape(-1).astype(jnp.int32)
    table2 = table.reshape(VOCAB // 2, 2 * EMB)
    pooled = _pool(x_flat, table2)
    return _matmul(pooled, W, b)


# speedup vs baseline: 1.2469x; 1.2469x over previous
import jax
import jax.numpy as jnp
from jax import lax
from jax.experimental import pallas as pl
from jax.experimental.pallas import tpu as pltpu
from jax.experimental.pallas import tpu_sc as plsc

VOCAB = 1000000
EMB = 64
OUT = 64
B = 4096
L = 200

NC = 2
NS = 16
NW = NC * NS
E_PER_W = B // NW
IDX_PER_W = E_PER_W * L

CHUNKS = ((0, 120), (120, 80))
UNROLL = 8

WIDEN_ROWS = 8000


def _widen_body(t_ref, o_ref):
    x = t_ref[...]
    o_ref[...] = jnp.concatenate([x, x], axis=1)


def _widen(table):
    return pl.pallas_call(
        _widen_body,
        grid=(VOCAB // WIDEN_ROWS,),
        in_specs=[pl.BlockSpec((WIDEN_ROWS, EMB), lambda i: (i, 0))],
        out_specs=pl.BlockSpec((WIDEN_ROWS, 2 * EMB), lambda i: (i, 0)),
        out_shape=jax.ShapeDtypeStruct((VOCAB, 2 * EMB), jnp.float32),
    )(table)


def _pool_body(x_hbm, t2_hbm, out_hbm, idx_v, rows_v, pooled_v, sem0, sem1):
    c = lax.axis_index("c")
    s = lax.axis_index("s")
    wid = s * NC + c
    base_e = wid * E_PER_W
    sems = (sem0, sem1)
    last = jnp.int32(E_PER_W - 1)

    pltpu.sync_copy(x_hbm.at[pl.ds(base_e * L, IDX_PER_W)], idx_v)

    def copies(e, buf):
        return [
            pltpu.make_async_copy(
                t2_hbm.at[idx_v.at[pl.ds(e * L + off, n)]],
                rows_v.at[buf].at[pl.ds(off, n)],
                sems[buf],
            )
            for off, n in CHUNKS
        ]

    def fire(e, buf):
        for cp in copies(e, buf):
            cp.start()

    def wait(buf):
        for cp in copies(jnp.int32(0), buf):
            cp.wait()

    def reduce_into(e, buf):
        def red(i, accs):
            r = i * UNROLL
            out = list(accs)
            for rr in range(UNROLL):
                for j in range(4):
                    out[j] = out[j] + rows_v[buf, r + rr, pl.ds(j * 16, 16)]
            return tuple(out)

        z = jnp.zeros((16,), jnp.float32)
        acc = lax.fori_loop(0, L // UNROLL, red, (z,) * 4, unroll=1)
        scale = jnp.float32(1.0 / L)
        for j in range(4):
            pooled_v[e, pl.ds(j * 16, 16)] = acc[j] * scale

    fire(jnp.int32(0), 0)
    fire(jnp.int32(1), 1)

    def pair(i, carry):
        e0 = 2 * i
        wait(0)
        reduce_into(e0, 0)
        fire(jnp.minimum(e0 + 2, last), 0)
        wait(1)
        reduce_into(e0 + 1, 1)
        fire(jnp.minimum(e0 + 3, last), 1)
        return carry

    lax.fori_loop(0, E_PER_W // 2, pair, 0)
    wait(0)
    wait(1)
    pltpu.sync_copy(pooled_v, out_hbm.at[pl.ds(base_e, E_PER_W)])


@jax.jit
def _pool(x_flat, table2):
    mesh = plsc.VectorSubcoreMesh(core_axis_name="c", subcore_axis_name="s")
    return pl.kernel(
        _pool_body,
        out_type=jax.ShapeDtypeStruct((B, EMB), jnp.float32),
        mesh=mesh,
        scratch_types=[
            pltpu.VMEM((IDX_PER_W,), jnp.int32),
            pltpu.VMEM((2, L, 2 * EMB), jnp.float32),
            pltpu.VMEM((E_PER_W, EMB), jnp.float32),
            pltpu.SemaphoreType.DMA,
            pltpu.SemaphoreType.DMA,
        ],
    )(x_flat, table2)


def _mm_body(p_ref, w_ref, b_ref, o_ref):
    o_ref[...] = (
        jnp.dot(p_ref[...], w_ref[...], preferred_element_type=jnp.float32)
        + b_ref[...]
    )


def _matmul(pooled, W, b):
    return pl.pallas_call(
        _mm_body,
        out_shape=jax.ShapeDtypeStruct((B, OUT), jnp.float32),
    )(pooled, W, b.reshape(1, OUT))


def kernel(x, table, W, b):
    x_flat = x.reshape(-1).astype(jnp.int32)
    table2 = _widen(table)
    pooled = _pool(x_flat, table2)
    return _matmul(pooled, W, b)


# speedup vs baseline: 2.0289x; 1.6272x over previous
import jax
import jax.numpy as jnp
from jax import lax
from jax.experimental import pallas as pl
from jax.experimental.pallas import tpu as pltpu
from jax.experimental.pallas import tpu_sc as plsc

VOCAB = 1000000
EMB = 64
OUT = 64
B = 4096
L = 200

NC = 2
NS = 16
NW = NC * NS
E_PER_W = B // NW
IDX_PER_W = E_PER_W * L

CHUNKS = ((0, 120), (120, 80))
UNROLL = 8

WIDEN_ROWS = 8192


def _widen_body(t_ref, o_ref):
    xt = t_ref[...].T
    o_ref[...] = jnp.concatenate([xt, xt], axis=1)


def _widen(table_t):
    return pl.pallas_call(
        _widen_body,
        grid=((VOCAB + WIDEN_ROWS - 1) // WIDEN_ROWS,),
        in_specs=[pl.BlockSpec((EMB, WIDEN_ROWS), lambda i: (0, i))],
        out_specs=pl.BlockSpec((WIDEN_ROWS, 2 * EMB), lambda i: (i, 0)),
        out_shape=jax.ShapeDtypeStruct((VOCAB, 2 * EMB), jnp.float32),
    )(table_t)


def _pool_body(x_hbm, t2_hbm, out_hbm, idx_v, rows_v, pooled_v, sem0, sem1):
    c = lax.axis_index("c")
    s = lax.axis_index("s")
    wid = s * NC + c
    base_e = wid * E_PER_W
    sems = (sem0, sem1)
    last = jnp.int32(E_PER_W - 1)

    pltpu.sync_copy(x_hbm.at[pl.ds(base_e * L, IDX_PER_W)], idx_v)

    def copies(e, buf):
        return [
            pltpu.make_async_copy(
                t2_hbm.at[idx_v.at[pl.ds(e * L + off, n)]],
                rows_v.at[buf].at[pl.ds(off, n)],
                sems[buf],
            )
            for off, n in CHUNKS
        ]

    def fire(e, buf):
        for cp in copies(e, buf):
            cp.start()

    def wait(buf):
        for cp in copies(jnp.int32(0), buf):
            cp.wait()

    def reduce_into(e, buf):
        def red(i, accs):
            r = i * UNROLL
            out = list(accs)
            for rr in range(UNROLL):
                for j in range(4):
                    out[j] = out[j] + rows_v[buf, r + rr, pl.ds(j * 16, 16)]
            return tuple(out)

        z = jnp.zeros((16,), jnp.float32)
        acc = lax.fori_loop(0, L // UNROLL, red, (z,) * 4, unroll=1)
        scale = jnp.float32(1.0 / L)
        for j in range(4):
            pooled_v[e, pl.ds(j * 16, 16)] = acc[j] * scale

    fire(jnp.int32(0), 0)
    fire(jnp.int32(1), 1)

    def pair(i, carry):
        e0 = 2 * i
        wait(0)
        reduce_into(e0, 0)
        fire(jnp.minimum(e0 + 2, last), 0)
        wait(1)
        reduce_into(e0 + 1, 1)
        fire(jnp.minimum(e0 + 3, last), 1)
        return carry

    lax.fori_loop(0, E_PER_W // 2, pair, 0)
    wait(0)
    wait(1)
    pltpu.sync_copy(pooled_v, out_hbm.at[pl.ds(base_e, E_PER_W)])


@jax.jit
def _pool(x_flat, table2):
    mesh = plsc.VectorSubcoreMesh(core_axis_name="c", subcore_axis_name="s")
    return pl.kernel(
        _pool_body,
        out_type=jax.ShapeDtypeStruct((B, EMB), jnp.float32),
        mesh=mesh,
        scratch_types=[
            pltpu.VMEM((IDX_PER_W,), jnp.int32),
            pltpu.VMEM((2, L, 2 * EMB), jnp.float32),
            pltpu.VMEM((E_PER_W, EMB), jnp.float32),
            pltpu.SemaphoreType.DMA,
            pltpu.SemaphoreType.DMA,
        ],
    )(x_flat, table2)


def _mm_body(p_ref, w_ref, b_ref, o_ref):
    o_ref[...] = (
        jnp.dot(p_ref[...], w_ref[...], preferred_element_type=jnp.float32)
        + b_ref[...]
    )


def _matmul(pooled, W, b):
    return pl.pallas_call(
        _mm_body,
        out_shape=jax.ShapeDtypeStruct((B, OUT), jnp.float32),
    )(pooled, W, b.reshape(1, OUT))


def kernel(x, table, W, b):
    x_flat = x.reshape(-1).astype(jnp.int32)
    table2 = _widen(table.T)
    pooled = _pool(x_flat, table2)
    return _matmul(pooled, W, b)
